# baseline (speedup 1.0000x reference)
import functools

import jax
import jax.numpy as jnp
from jax import lax
from jax.experimental import pallas as pl
from jax.experimental.pallas import tpu as pltpu
from jax.experimental.pallas import tpu_sc as plsc

N_NODES_C = 10000
N_EDGES_C = 320000
D_FEAT_C = 128
D_EDGE_C = 16
HID = 8

NC = 2
NS = 16
L = 16

N_PAIRS = NC * NS // 2
E_PER_PAIR = N_EDGES_C // N_PAIRS
SUB = 2000
NSUB = E_PER_PAIR // SUB


def _mm_abt_body(x_ref, w_ref, o_ref):
    o_ref[...] = lax.dot_general(
        w_ref[...], x_ref[...],
        dimension_numbers=(((0,), (1,)), ((), ())),
        preferred_element_type=jnp.float32,
        precision=lax.Precision.HIGHEST)


def _mm_c_body(ea_ref, w_ref, b_ref, o_ref):
    o_ref[...] = (
        jnp.dot(ea_ref[...], w_ref[...],
                preferred_element_type=jnp.float32,
                precision=lax.Precision.HIGHEST)
        + b_ref[...]
    )


_C_BLK = 4000


def _sc_edge_mlp(abt_hbm, ei_hbm, c_hbm, w2b_hbm,
                 out_hbm,
                 pa0, pa1, pa2, pa3, pb0, pb1, pb2, pb3,
                 rowv, colv, cv, sv, w2_v):
    c_id = lax.axis_index("c")
    s_id = lax.axis_index("s")
    pair = s_id // 2
    jh = s_id % 2
    ec = c_id * (NS // 2) + pair
    base = ec * E_PER_PAIR

    pa = [pa0, pa1, pa2, pa3]
    pb = [pb0, pb1, pb2, pb3]
    for jl in range(4):
        pltpu.sync_copy(abt_hbm.at[jh * 4 + jl], pa[jl])
        pltpu.sync_copy(abt_hbm.at[HID + jh * 4 + jl], pb[jl])
    pltpu.sync_copy(w2b_hbm.at[pl.ds(jh * 4, 4)], w2_v)

    w2r = [w2_v[j] for j in range(4)]
    zero16 = jnp.zeros((L,), jnp.float32)
    lane8 = lax.iota(jnp.int32, L) * 8

    def sub_chunk(k, carry):
        eb = base + k * SUB
        pltpu.sync_copy(ei_hbm.at[pl.ds(eb, SUB)], rowv)
        pltpu.sync_copy(ei_hbm.at[pl.ds(N_EDGES_C + eb, SUB)], colv)
        pltpu.sync_copy(c_hbm.at[pl.ds(eb * 8, SUB * 8)], cv)

        def inner(i, carry2):
            r16 = rowv[pl.ds(i * L, L)]
            q16 = colv[pl.ds(i * L, L)]
            cb = lane8 + i * (8 * L) + jh * 4
            s_acc = zero16
            for jl in range(4):
                ag = plsc.load_gather(pa[jl], [r16])
                bg = plsc.load_gather(pb[jl], [q16])
                cg = plsc.load_gather(cv, [cb + jl])
                pre = jnp.maximum(ag + bg + cg, 0.0)
                s_acc = s_acc + pre * w2r[jl]
            sv[pl.ds(i * L, L)] = s_acc
            return carry2

        lax.fori_loop(0, SUB // L, inner, 0)

        pltpu.sync_copy(sv, out_hbm.at[pl.ds(jh * N_EDGES_C + eb, SUB)])

        return carry

    lax.fori_loop(0, NSUB, sub_chunk, 0)


_sc_call = functools.partial(
    pl.kernel,
    out_type=jax.ShapeDtypeStruct((2 * N_EDGES_C,), jnp.float32),
    mesh=plsc.VectorSubcoreMesh(core_axis_name="c", subcore_axis_name="s"),
    compiler_params=pltpu.CompilerParams(needs_layout_passes=False),
    scratch_types=(
        [pltpu.VMEM((N_NODES_C,), jnp.float32)] * 8
        + [
            pltpu.VMEM((SUB,), jnp.int32),
            pltpu.VMEM((SUB,), jnp.int32),
            pltpu.VMEM((SUB * 8,), jnp.float32),
            pltpu.VMEM((SUB,), jnp.float32),
            pltpu.VMEM((4, L), jnp.float32),
        ]
    ),
)


def _combine_body(s0_ref, s1_ref, b2_ref, o_ref):
    v = s0_ref[...] + s1_ref[...] + b2_ref[...]
    o_ref[...] = jax.nn.sigmoid(v)


def kernel(xn, edge_index, edge_attr, W1, b1, W2, b2):
    W1a = W1[:D_FEAT_C, :]
    W1b = W1[D_FEAT_C:2 * D_FEAT_C, :]
    W1c = W1[2 * D_FEAT_C:, :]
    Wab = jnp.concatenate([W1a, W1b], axis=1)

    ABt = pl.pallas_call(
        _mm_abt_body,
        out_shape=jax.ShapeDtypeStruct((2 * HID, N_NODES_C), jnp.float32),
    )(xn, Wab)

    ea_pack = edge_attr.reshape(N_EDGES_C // 8, 8 * D_EDGE_C)
    eye8 = jnp.eye(8, dtype=jnp.float32)
    w_bd = jnp.einsum("pq,kh->pkqh", eye8, W1c).reshape(8 * D_EDGE_C, 8 * HID)
    b_tile = jnp.tile(b1, 8).reshape(1, 8 * HID)
    C = pl.pallas_call(
        _mm_c_body,
        grid=(N_EDGES_C // 8 // _C_BLK,),
        in_specs=[
            pl.BlockSpec((_C_BLK, 8 * D_EDGE_C), lambda i: (i, 0)),
            pl.BlockSpec((8 * D_EDGE_C, 8 * HID), lambda i: (0, 0)),
            pl.BlockSpec((1, 8 * HID), lambda i: (0, 0)),
        ],
        out_specs=pl.BlockSpec((_C_BLK, 8 * HID), lambda i: (i, 0)),
        out_shape=jax.ShapeDtypeStruct((N_EDGES_C // 8, 8 * HID), jnp.float32),
    )(ea_pack, w_bd, b_tile)

    ei32 = edge_index.astype(jnp.int32).reshape(2 * N_EDGES_C)
    c_flat = C.reshape(-1)
    w2b = jnp.broadcast_to(W2.reshape(HID, 1), (HID, L))

    s01 = _sc_call(_sc_edge_mlp)(ABt, ei32, c_flat, w2b)
    s01_2d = s01.reshape(80, 8000)

    out = pl.pallas_call(
        _combine_body,
        grid=(1,),
        in_specs=[
            pl.BlockSpec((40, 8000), lambda i: (0, 0)),
            pl.BlockSpec((40, 8000), lambda i: (1, 0)),
            pl.BlockSpec((1, 1), lambda i: (0, 0)),
        ],
        out_specs=pl.BlockSpec((40, 8000), lambda i: (0, 0)),
        out_shape=jax.ShapeDtypeStruct((40, 8000), jnp.float32),
    )(s01_2d, s01_2d, b2.reshape(1, 1))
    return out.reshape(N_EDGES_C, 1)


# speedup vs baseline: 1.0375x; 1.0375x over previous
import functools

import jax
import jax.numpy as jnp
from jax import lax
from jax.experimental import pallas as pl
from jax.experimental.pallas import tpu as pltpu
from jax.experimental.pallas import tpu_sc as plsc

N_NODES_C = 10000
N_EDGES_C = 320000
D_FEAT_C = 128
D_EDGE_C = 16
HID = 8

NC = 2
NS = 16
L = 16

N_PAIRS = NC * NS // 2
E_PER_PAIR = N_EDGES_C // N_PAIRS
SUB = 2000
NSUB = E_PER_PAIR // SUB


def _mm_abt_body(x_ref, w_ref, o_ref):
    o_ref[...] = lax.dot_general(
        w_ref[...], x_ref[...],
        dimension_numbers=(((0,), (1,)), ((), ())),
        preferred_element_type=jnp.float32,
        precision=lax.Precision.HIGHEST)


def _mm_c_body(ea_ref, w_ref, b_ref, o_ref):
    o_ref[...] = (
        jnp.dot(ea_ref[...], w_ref[...],
                preferred_element_type=jnp.float32,
                precision=lax.Precision.HIGHEST)
        + b_ref[...]
    )


_C_BLK = 4000


def _sc_edge_mlp(abt_hbm, ei_hbm, c_hbm, w2b_hbm,
                 out_hbm,
                 pa0, pa1, pa2, pa3, pb0, pb1, pb2, pb3,
                 rowv, colv, cv, sv, w2_v):
    c_id = lax.axis_index("c")
    s_id = lax.axis_index("s")
    pair = s_id // 2
    jh = s_id % 2
    ec = c_id * (NS // 2) + pair
    base = ec * E_PER_PAIR

    pa = [pa0, pa1, pa2, pa3]
    pb = [pb0, pb1, pb2, pb3]
    for jl in range(4):
        pltpu.sync_copy(abt_hbm.at[jh * 4 + jl], pa[jl])
        pltpu.sync_copy(abt_hbm.at[HID + jh * 4 + jl], pb[jl])
    pltpu.sync_copy(w2b_hbm.at[pl.ds(jh * 4, 4)], w2_v)

    w2r = [w2_v[j] for j in range(4)]
    zero16 = jnp.zeros((L,), jnp.float32)
    lane8 = lax.iota(jnp.int32, L) * 8

    def sub_chunk(k, carry):
        eb = base + k * SUB
        pltpu.sync_copy(ei_hbm.at[pl.ds(eb, SUB)], rowv)
        pltpu.sync_copy(ei_hbm.at[pl.ds(N_EDGES_C + eb, SUB)], colv)
        pltpu.sync_copy(c_hbm.at[pl.ds(eb * 8, SUB * 8)], cv)

        @plsc.parallel_loop(0, SUB // L, unroll=8)
        def inner(i):
            r16 = rowv[pl.ds(i * L, L)]
            q16 = colv[pl.ds(i * L, L)]
            cb = lane8 + i * (8 * L) + jh * 4
            s_acc = zero16
            for jl in range(4):
                ag = plsc.load_gather(pa[jl], [r16])
                bg = plsc.load_gather(pb[jl], [q16])
                cg = plsc.load_gather(cv, [cb + jl])
                pre = jnp.maximum(ag + bg + cg, 0.0)
                s_acc = s_acc + pre * w2r[jl]
            sv[pl.ds(i * L, L)] = s_acc

        pltpu.sync_copy(sv, out_hbm.at[pl.ds(jh * N_EDGES_C + eb, SUB)])

        return carry

    lax.fori_loop(0, NSUB, sub_chunk, 0)


_sc_call = functools.partial(
    pl.kernel,
    out_type=jax.ShapeDtypeStruct((2 * N_EDGES_C,), jnp.float32),
    mesh=plsc.VectorSubcoreMesh(core_axis_name="c", subcore_axis_name="s"),
    compiler_params=pltpu.CompilerParams(needs_layout_passes=False),
    scratch_types=(
        [pltpu.VMEM((N_NODES_C,), jnp.float32)] * 8
        + [
            pltpu.VMEM((SUB,), jnp.int32),
            pltpu.VMEM((SUB,), jnp.int32),
            pltpu.VMEM((SUB * 8,), jnp.float32),
            pltpu.VMEM((SUB,), jnp.float32),
            pltpu.VMEM((4, L), jnp.float32),
        ]
    ),
)


def _combine_body(s0_ref, s1_ref, b2_ref, o_ref):
    v = s0_ref[...] + s1_ref[...] + b2_ref[...]
    o_ref[...] = jax.nn.sigmoid(v)


def kernel(xn, edge_index, edge_attr, W1, b1, W2, b2):
    W1a = W1[:D_FEAT_C, :]
    W1b = W1[D_FEAT_C:2 * D_FEAT_C, :]
    W1c = W1[2 * D_FEAT_C:, :]
    Wab = jnp.concatenate([W1a, W1b], axis=1)

    ABt = pl.pallas_call(
        _mm_abt_body,
        out_shape=jax.ShapeDtypeStruct((2 * HID, N_NODES_C), jnp.float32),
    )(xn, Wab)

    ea_pack = edge_attr.reshape(N_EDGES_C // 8, 8 * D_EDGE_C)
    eye8 = jnp.eye(8, dtype=jnp.float32)
    w_bd = jnp.einsum("pq,kh->pkqh", eye8, W1c).reshape(8 * D_EDGE_C, 8 * HID)
    b_tile = jnp.tile(b1, 8).reshape(1, 8 * HID)
    C = pl.pallas_call(
        _mm_c_body,
        grid=(N_EDGES_C // 8 // _C_BLK,),
        in_specs=[
            pl.BlockSpec((_C_BLK, 8 * D_EDGE_C), lambda i: (i, 0)),
            pl.BlockSpec((8 * D_EDGE_C, 8 * HID), lambda i: (0, 0)),
            pl.BlockSpec((1, 8 * HID), lambda i: (0, 0)),
        ],
        out_specs=pl.BlockSpec((_C_BLK, 8 * HID), lambda i: (i, 0)),
        out_shape=jax.ShapeDtypeStruct((N_EDGES_C // 8, 8 * HID), jnp.float32),
    )(ea_pack, w_bd, b_tile)

    ei32 = edge_index.astype(jnp.int32).reshape(2 * N_EDGES_C)
    c_flat = C.reshape(-1)
    w2b = jnp.broadcast_to(W2.reshape(HID, 1), (HID, L))

    s01 = _sc_call(_sc_edge_mlp)(ABt, ei32, c_flat, w2b)
    s01_2d = s01.reshape(80, 8000)

    out = pl.pallas_call(
        _combine_body,
        grid=(1,),
        in_specs=[
            pl.BlockSpec((40, 8000), lambda i: (0, 0)),
            pl.BlockSpec((40, 8000), lambda i: (1, 0)),
            pl.BlockSpec((1, 1), lambda i: (0, 0)),
        ],
        out_specs=pl.BlockSpec((40, 8000), lambda i: (0, 0)),
        out_shape=jax.ShapeDtypeStruct((40, 8000), jnp.float32),
    )(s01_2d, s01_2d, b2.reshape(1, 1))
    return out.reshape(N_EDGES_C, 1)


# speedup vs baseline: 1.0438x; 1.0060x over previous
import functools

import jax
import jax.numpy as jnp
from jax import lax
from jax.experimental import pallas as pl
from jax.experimental.pallas import tpu as pltpu
from jax.experimental.pallas import tpu_sc as plsc

N_NODES_C = 10000
N_EDGES_C = 320000
D_FEAT_C = 128
D_EDGE_C = 16
HID = 8

NC = 2
NS = 16
L = 16

N_PAIRS = NC * NS // 2
E_PER_PAIR = N_EDGES_C // N_PAIRS
SUB = 2000
NSUB = E_PER_PAIR // SUB


def _mm_abt_body(x_ref, w_ref, o_ref):
    o_ref[...] = lax.dot_general(
        w_ref[...], x_ref[...],
        dimension_numbers=(((0,), (1,)), ((), ())),
        preferred_element_type=jnp.float32,
        precision=lax.Precision.HIGHEST)


def _mm_c_body(ea_ref, w_ref, b_ref, o_ref):
    o_ref[...] = (
        jnp.dot(ea_ref[...], w_ref[...],
                preferred_element_type=jnp.float32,
                precision=lax.Precision.HIGHEST)
        + b_ref[...]
    )


_C_BLK = 4000


def _sc_edge_mlp(abt_hbm, ei_hbm, c_hbm, w2b_hbm,
                 out_hbm,
                 pa0, pa1, pa2, pa3, pb0, pb1, pb2, pb3,
                 rowv, colv, cv, sv, w2_v):
    c_id = lax.axis_index("c")
    s_id = lax.axis_index("s")
    pair = s_id // 2
    jh = s_id % 2
    ec = c_id * (NS // 2) + pair
    base = ec * E_PER_PAIR

    pa = [pa0, pa1, pa2, pa3]
    pb = [pb0, pb1, pb2, pb3]
    for jl in range(4):
        pltpu.sync_copy(abt_hbm.at[jh * 4 + jl], pa[jl])
        pltpu.sync_copy(abt_hbm.at[HID + jh * 4 + jl], pb[jl])
    pltpu.sync_copy(w2b_hbm.at[pl.ds(jh * 4, 4)], w2_v)

    w2r = [w2_v[j] for j in range(4)]
    zero16 = jnp.zeros((L,), jnp.float32)
    lane8 = lax.iota(jnp.int32, L) * 8

    def sub_chunk(k, carry):
        eb = base + k * SUB
        pltpu.sync_copy(ei_hbm.at[pl.ds(eb, SUB)], rowv)
        pltpu.sync_copy(ei_hbm.at[pl.ds(N_EDGES_C + eb, SUB)], colv)
        pltpu.sync_copy(c_hbm.at[pl.ds(eb * 8, SUB * 8)], cv)

        @plsc.parallel_loop(0, SUB // L, unroll=4)
        def inner(i):
            r16 = rowv[pl.ds(i * L, L)]
            q16 = colv[pl.ds(i * L, L)]
            cb = lane8 + i * (8 * L) + jh * 4
            s_acc = zero16
            for jl in range(4):
                ag = plsc.load_gather(pa[jl], [r16])
                bg = plsc.load_gather(pb[jl], [q16])
                cg = plsc.load_gather(cv, [cb + jl])
                pre = jnp.maximum(ag + bg + cg, 0.0)
                s_acc = s_acc + pre * w2r[jl]
            sv[pl.ds(i * L, L)] = s_acc

        pltpu.sync_copy(sv, out_hbm.at[pl.ds(jh * N_EDGES_C + eb, SUB)])

        return carry

    lax.fori_loop(0, NSUB, sub_chunk, 0)


_sc_call = functools.partial(
    pl.kernel,
    out_type=jax.ShapeDtypeStruct((2 * N_EDGES_C,), jnp.float32),
    mesh=plsc.VectorSubcoreMesh(core_axis_name="c", subcore_axis_name="s"),
    compiler_params=pltpu.CompilerParams(needs_layout_passes=False),
    scratch_types=(
        [pltpu.VMEM((N_NODES_C,), jnp.float32)] * 8
        + [
            pltpu.VMEM((SUB,), jnp.int32),
            pltpu.VMEM((SUB,), jnp.int32),
            pltpu.VMEM((SUB * 8,), jnp.float32),
            pltpu.VMEM((SUB,), jnp.float32),
            pltpu.VMEM((4, L), jnp.float32),
        ]
    ),
)


def _combine_body(s0_ref, s1_ref, b2_ref, o_ref):
    v = s0_ref[...] + s1_ref[...] + b2_ref[...]
    o_ref[...] = jax.nn.sigmoid(v)


def kernel(xn, edge_index, edge_attr, W1, b1, W2, b2):
    W1a = W1[:D_FEAT_C, :]
    W1b = W1[D_FEAT_C:2 * D_FEAT_C, :]
    W1c = W1[2 * D_FEAT_C:, :]
    Wab = jnp.concatenate([W1a, W1b], axis=1)

    ABt = pl.pallas_call(
        _mm_abt_body,
        out_shape=jax.ShapeDtypeStruct((2 * HID, N_NODES_C), jnp.float32),
    )(xn, Wab)

    ea_pack = edge_attr.reshape(N_EDGES_C // 8, 8 * D_EDGE_C)
    eye8 = jnp.eye(8, dtype=jnp.float32)
    w_bd = jnp.einsum("pq,kh->pkqh", eye8, W1c).reshape(8 * D_EDGE_C, 8 * HID)
    b_tile = jnp.tile(b1, 8).reshape(1, 8 * HID)
    C = pl.pallas_call(
        _mm_c_body,
        grid=(N_EDGES_C // 8 // _C_BLK,),
        in_specs=[
            pl.BlockSpec((_C_BLK, 8 * D_EDGE_C), lambda i: (i, 0)),
            pl.BlockSpec((8 * D_EDGE_C, 8 * HID), lambda i: (0, 0)),
            pl.BlockSpec((1, 8 * HID), lambda i: (0, 0)),
        ],
        out_specs=pl.BlockSpec((_C_BLK, 8 * HID), lambda i: (i, 0)),
        out_shape=jax.ShapeDtypeStruct((N_EDGES_C // 8, 8 * HID), jnp.float32),
    )(ea_pack, w_bd, b_tile)

    ei32 = edge_index.astype(jnp.int32).reshape(2 * N_EDGES_C)
    c_flat = C.reshape(-1)
    w2b = jnp.broadcast_to(W2.reshape(HID, 1), (HID, L))

    s01 = _sc_call(_sc_edge_mlp)(ABt, ei32, c_flat, w2b)
    s01_2d = s01.reshape(80, 8000)

    out = pl.pallas_call(
        _combine_body,
        grid=(1,),
        in_specs=[
            pl.BlockSpec((40, 8000), lambda i: (0, 0)),
            pl.BlockSpec((40, 8000), lambda i: (1, 0)),
            pl.BlockSpec((1, 1), lambda i: (0, 0)),
        ],
        out_specs=pl.BlockSpec((40, 8000), lambda i: (0, 0)),
        out_shape=jax.ShapeDtypeStruct((40, 8000), jnp.float32),
    )(s01_2d, s01_2d, b2.reshape(1, 1))
    return out.reshape(N_EDGES_C, 1)


# speedup vs baseline: 1.0815x; 1.0362x over previous
import functools

import jax
import jax.numpy as jnp
from jax import lax
from jax.experimental import pallas as pl
from jax.experimental.pallas import tpu as pltpu
from jax.experimental.pallas import tpu_sc as plsc

N_NODES_C = 10000
N_EDGES_C = 320000
D_FEAT_C = 128
D_EDGE_C = 16
HID = 8

NC = 2
NS = 16
L = 16

N_PAIRS = NC * NS // 2
E_PER_PAIR = N_EDGES_C // N_PAIRS
SUB = 4000
NSUB = E_PER_PAIR // SUB


def _mm_abt_body(x_ref, w_ref, o_ref):
    o_ref[...] = lax.dot_general(
        w_ref[...], x_ref[...],
        dimension_numbers=(((0,), (1,)), ((), ())),
        preferred_element_type=jnp.float32,
        precision=lax.Precision.HIGHEST)


def _mm_c_body(ea_ref, w_ref, b_ref, o_ref):
    o_ref[...] = (
        jnp.dot(ea_ref[...], w_ref[...],
                preferred_element_type=jnp.float32,
                precision=lax.Precision.HIGHEST)
        + b_ref[...]
    )


_C_BLK = 4000


def _sc_edge_mlp(abt_hbm, ei_hbm, c_hbm, w2b_hbm,
                 out_hbm,
                 pa0, pa1, pa2, pa3, pb0, pb1, pb2, pb3,
                 rowv, colv, cv, sv, w2_v):
    c_id = lax.axis_index("c")
    s_id = lax.axis_index("s")
    pair = s_id // 2
    jh = s_id % 2
    ec = c_id * (NS // 2) + pair
    base = ec * E_PER_PAIR

    pa = [pa0, pa1, pa2, pa3]
    pb = [pb0, pb1, pb2, pb3]
    for jl in range(4):
        pltpu.sync_copy(abt_hbm.at[jh * 4 + jl], pa[jl])
        pltpu.sync_copy(abt_hbm.at[HID + jh * 4 + jl], pb[jl])
    pltpu.sync_copy(w2b_hbm.at[pl.ds(jh * 4, 4)], w2_v)

    w2r = [w2_v[j] for j in range(4)]
    zero16 = jnp.zeros((L,), jnp.float32)
    lane8 = lax.iota(jnp.int32, L) * 8

    def sub_chunk(k, carry):
        eb = base + k * SUB
        pltpu.sync_copy(ei_hbm.at[pl.ds(eb, SUB)], rowv)
        pltpu.sync_copy(ei_hbm.at[pl.ds(N_EDGES_C + eb, SUB)], colv)
        pltpu.sync_copy(c_hbm.at[pl.ds(eb * 8, SUB * 8)], cv)

        @plsc.parallel_loop(0, SUB // L, unroll=4)
        def inner(i):
            r16 = rowv[pl.ds(i * L, L)]
            q16 = colv[pl.ds(i * L, L)]
            cb = lane8 + i * (8 * L) + jh * 4
            s_acc = zero16
            for jl in range(4):
                ag = plsc.load_gather(pa[jl], [r16])
                bg = plsc.load_gather(pb[jl], [q16])
                cg = plsc.load_gather(cv, [cb + jl])
                pre = jnp.maximum(ag + bg + cg, 0.0)
                s_acc = s_acc + pre * w2r[jl]
            sv[pl.ds(i * L, L)] = s_acc

        pltpu.sync_copy(sv, out_hbm.at[pl.ds(jh * N_EDGES_C + eb, SUB)])

        return carry

    lax.fori_loop(0, NSUB, sub_chunk, 0)


_sc_call = functools.partial(
    pl.kernel,
    out_type=jax.ShapeDtypeStruct((2 * N_EDGES_C,), jnp.float32),
    mesh=plsc.VectorSubcoreMesh(core_axis_name="c", subcore_axis_name="s"),
    compiler_params=pltpu.CompilerParams(needs_layout_passes=False),
    scratch_types=(
        [pltpu.VMEM((N_NODES_C,), jnp.float32)] * 8
        + [
            pltpu.VMEM((SUB,), jnp.int32),
            pltpu.VMEM((SUB,), jnp.int32),
            pltpu.VMEM((SUB * 8,), jnp.float32),
            pltpu.VMEM((SUB,), jnp.float32),
            pltpu.VMEM((4, L), jnp.float32),
        ]
    ),
)


def _combine_body(s0_ref, s1_ref, b2_ref, o_ref):
    v = s0_ref[...] + s1_ref[...] + b2_ref[...]
    o_ref[...] = jax.nn.sigmoid(v)


def kernel(xn, edge_index, edge_attr, W1, b1, W2, b2):
    W1a = W1[:D_FEAT_C, :]
    W1b = W1[D_FEAT_C:2 * D_FEAT_C, :]
    W1c = W1[2 * D_FEAT_C:, :]
    Wab = jnp.concatenate([W1a, W1b], axis=1)

    ABt = pl.pallas_call(
        _mm_abt_body,
        out_shape=jax.ShapeDtypeStruct((2 * HID, N_NODES_C), jnp.float32),
    )(xn, Wab)

    ea_pack = edge_attr.reshape(N_EDGES_C // 8, 8 * D_EDGE_C)
    eye8 = jnp.eye(8, dtype=jnp.float32)
    w_bd = jnp.einsum("pq,kh->pkqh", eye8, W1c).reshape(8 * D_EDGE_C, 8 * HID)
    b_tile = jnp.tile(b1, 8).reshape(1, 8 * HID)
    C = pl.pallas_call(
        _mm_c_body,
        grid=(N_EDGES_C // 8 // _C_BLK,),
        in_specs=[
            pl.BlockSpec((_C_BLK, 8 * D_EDGE_C), lambda i: (i, 0)),
            pl.BlockSpec((8 * D_EDGE_C, 8 * HID), lambda i: (0, 0)),
            pl.BlockSpec((1, 8 * HID), lambda i: (0, 0)),
        ],
        out_specs=pl.BlockSpec((_C_BLK, 8 * HID), lambda i: (i, 0)),
        out_shape=jax.ShapeDtypeStruct((N_EDGES_C // 8, 8 * HID), jnp.float32),
    )(ea_pack, w_bd, b_tile)

    ei32 = edge_index.astype(jnp.int32).reshape(2 * N_EDGES_C)
    c_flat = C.reshape(-1)
    w2b = jnp.broadcast_to(W2.reshape(HID, 1), (HID, L))

    s01 = _sc_call(_sc_edge_mlp)(ABt, ei32, c_flat, w2b)
    s01_2d = s01.reshape(80, 8000)

    out = pl.pallas_call(
        _combine_body,
        grid=(1,),
        in_specs=[
            pl.BlockSpec((40, 8000), lambda i: (0, 0)),
            pl.BlockSpec((40, 8000), lambda i: (1, 0)),
            pl.BlockSpec((1, 1), lambda i: (0, 0)),
        ],
        out_specs=pl.BlockSpec((40, 8000), lambda i: (0, 0)),
        out_shape=jax.ShapeDtypeStruct((40, 8000), jnp.float32),
    )(s01_2d, s01_2d, b2.reshape(1, 1))
    return out.reshape(N_EDGES_C, 1)
